# Initial kernel scaffold; baseline (speedup 1.0000x reference)
import jax
import jax.numpy as jnp
from jax.experimental import pallas as pl


def kernel(rois_A, cls_prob_A, bbox_pred_A, im_info_A, rois_B, cls_prob_B, bbox_pred_B, im_info_B):
    raise NotImplementedError("write your pallas kernel here")


# speedup vs baseline: 7.7032x; 7.7032x over previous
import functools

import jax
import jax.numpy as jnp
from jax import lax
from jax.experimental import pallas as pl
import jax.experimental.pallas.tpu as pltpu

_THRESH = 0.05
_K = 50
_N = 20000
_C = 21
_NCLS = _C - 1
_FLAT = _N * _NCLS
_SEG = 128
_NSEG = _FLAT // _SEG
_SR = 25
_SC = 125
_BIG = 2**30
_NEG = -3.0


def _select_one(cls3, flat_ref, rois_ref, bbox_ref, wm, hm,
                cand_ref, cidx_ref, fvec_ref, svec_ref, g_ref, r_ref):
    masked3 = jnp.where(cls3 > _THRESH, cls3, -1.0)
    segmax = jnp.max(masked3, axis=2)
    seg_id = (lax.broadcasted_iota(jnp.int32, (_SR, _SC), 0) * _SC
              + lax.broadcasted_iota(jnp.int32, (_SR, _SC), 1))
    lane = lax.broadcasted_iota(jnp.int32, (1, _SEG), 1)

    def body1(i, sm):
        v = jnp.max(sm)
        s = jnp.min(jnp.where(sm == v, seg_id, _BIG))
        row = flat_ref[pl.ds(s, 1), :]
        cand_ref[pl.ds(i, 1), :] = jnp.where(row > _THRESH, row, -1.0)
        cidx_ref[pl.ds(i, 1), :] = s * _SEG + lane
        return jnp.where(seg_id == s, _NEG, sm)

    lax.fori_loop(0, _K, body1, segmax)

    cand0 = cand_ref[...]
    cidx = cidx_ref[...]

    def body2(i, c):
        v = jnp.max(c)
        f = jnp.min(jnp.where(c == v, cidx, _BIG))
        rrow = f // _NCLS
        svec_ref[pl.ds(i, 1), :] = jnp.full((1, 1), v, jnp.float32)
        fvec_ref[pl.ds(i, 1), :] = jnp.full((1, 1), f, jnp.int32)
        g_ref[pl.ds(i, 1), :] = bbox_ref[pl.ds(rrow, 1), :]
        r_ref[pl.ds(i, 1), :] = rois_ref[pl.ds(rrow, 1), :]
        return jnp.where(cidx == f, _NEG, c)

    lax.fori_loop(0, _K, body2, cand0)

    f = fvec_ref[...]
    cls = f - (f // _NCLS) * _NCLS
    G = g_ref[...]
    R = r_ref[...]
    sv = svec_ref[...]

    l84 = lax.broadcasted_iota(jnp.int32, (1, 4 * _C), 1)
    base = (cls + 1) * 4

    def ext(k):
        return jnp.sum(jnp.where(l84 == base + k, G, 0.0), axis=1, keepdims=True)

    d0, d1, d2, d3 = ext(0), ext(1), ext(2), ext(3)
    x1 = R[:, 1:2]
    y1 = R[:, 2:3]
    x2 = R[:, 3:4]
    y2 = R[:, 4:5]
    w = x2 - x1 + 1.0
    h = y2 - y1 + 1.0
    cx = x1 + 0.5 * w
    cy = y1 + 0.5 * h
    pcx = d0 * w + cx
    pcy = d1 * h + cy
    pw = jnp.exp(d2) * w
    ph = jnp.exp(d3) * h
    px1 = jnp.clip(pcx - 0.5 * pw, 0.0, wm)
    py1 = jnp.clip(pcy - 0.5 * ph, 0.0, hm)
    px2 = jnp.clip(pcx + 0.5 * pw - 1.0, 0.0, wm)
    py2 = jnp.clip(pcy + 0.5 * ph - 1.0, 0.0, hm)
    return px1, py1, px2, py2, sv


def _t(col):
    si = lax.broadcasted_iota(jnp.int32, (_K, _K), 0)
    li = lax.broadcasted_iota(jnp.int32, (_K, _K), 1)
    return jnp.sum(jnp.where(si == li, col, 0.0), axis=0, keepdims=True)


def _kern(cls3_a, flat_a, rois_a, bbox_a, im_a,
          cls3_b, flat_b, rois_b, bbox_b, im_b,
          out_a, out_b, out_idx,
          cand_ref, cidx_ref, fvec_ref, svec_ref, g_ref, r_ref):
    wm_a = im_a[0, 1] - 1.0
    hm_a = im_a[0, 0] - 1.0
    ax1, ay1, ax2, ay2, asv = _select_one(
        cls3_a[...], flat_a, rois_a, bbox_a, wm_a, hm_a,
        cand_ref, cidx_ref, fvec_ref, svec_ref, g_ref, r_ref)

    wm_b = im_b[0, 1] - 1.0
    hm_b = im_b[0, 0] - 1.0
    bx1, by1, bx2, by2, bsv = _select_one(
        cls3_b[...], flat_b, rois_b, bbox_b, wm_b, hm_b,
        cand_ref, cidx_ref, fvec_ref, svec_ref, g_ref, r_ref)

    bx1t, by1t, bx2t, by2t = _t(bx1), _t(by1), _t(bx2), _t(by2)
    area_a = (ax2 - ax1 + 1.0) * (ay2 - ay1 + 1.0)
    area_bt = (bx2t - bx1t + 1.0) * (by2t - by1t + 1.0)

    ltx = jnp.maximum(ax1, bx1t)
    lty = jnp.maximum(ay1, by1t)
    rbx = jnp.minimum(ax2, bx2t)
    rby = jnp.minimum(ay2, by2t)
    wi = jnp.maximum(rbx - ltx + 1.0, 0.0)
    hi = jnp.maximum(rby - lty + 1.0, 0.0)
    inter = wi * hi
    iou = inter / (area_a + area_bt - inter)

    m = jnp.max(iou, axis=1, keepdims=True)
    l50 = lax.broadcasted_iota(jnp.int32, (_K, _K), 1)
    idx = jnp.min(jnp.where(iou == m, l50, _BIG), axis=1, keepdims=True)

    box_b = jnp.concatenate([bx1, by1, bx2, by2, bsv], axis=1)
    oh = (l50 == idx).astype(jnp.float32)
    out_a[...] = jnp.concatenate([ax1, ay1, ax2, ay2, asv], axis=1)
    out_b[...] = jnp.dot(oh, box_b, preferred_element_type=jnp.float32)
    out_idx[...] = idx


@functools.partial(jax.jit, static_argnames=())
def _run(rois_a, cls_a, bbox_a, im_a, rois_b, cls_b, bbox_b, im_b):
    sa = cls_a[:, 1:]
    sb = cls_b[:, 1:]
    cls3_a = sa.reshape(_SR, _SC, _SEG)
    flat_a = sa.reshape(_NSEG, _SEG)
    cls3_b = sb.reshape(_SR, _SC, _SEG)
    flat_b = sb.reshape(_NSEG, _SEG)

    vspec = pl.BlockSpec(memory_space=pltpu.VMEM)
    sspec = pl.BlockSpec(memory_space=pltpu.SMEM)
    out = pl.pallas_call(
        _kern,
        out_shape=[
            jax.ShapeDtypeStruct((_K, 5), jnp.float32),
            jax.ShapeDtypeStruct((_K, 5), jnp.float32),
            jax.ShapeDtypeStruct((_K, 1), jnp.int32),
        ],
        in_specs=[vspec, vspec, vspec, vspec, sspec,
                  vspec, vspec, vspec, vspec, sspec],
        out_specs=[vspec, vspec, vspec],
        scratch_shapes=[
            pltpu.VMEM((_K, _SEG), jnp.float32),
            pltpu.VMEM((_K, _SEG), jnp.int32),
            pltpu.VMEM((_K, 1), jnp.int32),
            pltpu.VMEM((_K, 1), jnp.float32),
            pltpu.VMEM((_K, 4 * _C), jnp.float32),
            pltpu.VMEM((_K, 5), jnp.float32),
        ],
    )(cls3_a, flat_a, rois_a, bbox_a, im_a,
      cls3_b, flat_b, rois_b, bbox_b, im_b)
    box_a, box_b, idx = out
    return box_a, box_b, idx.reshape(_K)


def kernel(rois_A, cls_prob_A, bbox_pred_A, im_info_A,
           rois_B, cls_prob_B, bbox_pred_B, im_info_B):
    return _run(rois_A, cls_prob_A, bbox_pred_A, im_info_A,
                rois_B, cls_prob_B, bbox_pred_B, im_info_B)


# speedup vs baseline: 7.7581x; 1.0071x over previous
import functools

import jax
import jax.numpy as jnp
from jax import lax
from jax.experimental import pallas as pl
import jax.experimental.pallas.tpu as pltpu

_THRESH = 0.05
_K = 50
_N = 20000
_C = 21
_NCLS = _C - 1
_FLAT = _N * _NCLS
_SEG = 128
_NSEG = _FLAT // _SEG
_SR = 25
_SC = 125
_BIG = 2**30
_NEG = -3.0


def _select_one(cls3, flat_ref, rois_ref, bbox_ref, wm, hm,
                cand_ref, cidx_ref, fvec_ref, svec_ref, g_ref, r_ref):
    masked3 = jnp.where(cls3 > _THRESH, cls3, -1.0)
    segmax = jnp.max(masked3, axis=2)
    seg_id = (lax.broadcasted_iota(jnp.int32, (_SR, _SC), 0) * _SC
              + lax.broadcasted_iota(jnp.int32, (_SR, _SC), 1))
    lane = lax.broadcasted_iota(jnp.int32, (1, _SEG), 1)

    sm = segmax
    for i in range(_K):
        v = jnp.max(sm)
        s = jnp.min(jnp.where(sm == v, seg_id, _BIG))
        row = flat_ref[pl.ds(s, 1), :]
        cand_ref[i:i + 1, :] = jnp.where(row > _THRESH, row, -1.0)
        cidx_ref[i:i + 1, :] = s * _SEG + lane
        sm = jnp.where(seg_id == s, _NEG, sm)

    c = cand_ref[...]
    cidx = cidx_ref[...]

    for i in range(_K):
        v = jnp.max(c)
        f = jnp.min(jnp.where(c == v, cidx, _BIG))
        rrow = f // _NCLS
        svec_ref[i:i + 1, :] = jnp.full((1, 1), v, jnp.float32)
        fvec_ref[i:i + 1, :] = jnp.full((1, 1), f, jnp.int32)
        g_ref[i:i + 1, :] = bbox_ref[pl.ds(rrow, 1), :]
        r_ref[i:i + 1, :] = rois_ref[pl.ds(rrow, 1), :]
        c = jnp.where(cidx == f, _NEG, c)

    f = fvec_ref[...]
    cls = f - (f // _NCLS) * _NCLS
    G = g_ref[...]
    R = r_ref[...]
    sv = svec_ref[...]

    l84 = lax.broadcasted_iota(jnp.int32, (1, 4 * _C), 1)
    base = (cls + 1) * 4

    def ext(k):
        return jnp.sum(jnp.where(l84 == base + k, G, 0.0), axis=1, keepdims=True)

    d0, d1, d2, d3 = ext(0), ext(1), ext(2), ext(3)
    x1 = R[:, 1:2]
    y1 = R[:, 2:3]
    x2 = R[:, 3:4]
    y2 = R[:, 4:5]
    w = x2 - x1 + 1.0
    h = y2 - y1 + 1.0
    cx = x1 + 0.5 * w
    cy = y1 + 0.5 * h
    pcx = d0 * w + cx
    pcy = d1 * h + cy
    pw = jnp.exp(d2) * w
    ph = jnp.exp(d3) * h
    px1 = jnp.clip(pcx - 0.5 * pw, 0.0, wm)
    py1 = jnp.clip(pcy - 0.5 * ph, 0.0, hm)
    px2 = jnp.clip(pcx + 0.5 * pw - 1.0, 0.0, wm)
    py2 = jnp.clip(pcy + 0.5 * ph - 1.0, 0.0, hm)
    return px1, py1, px2, py2, sv


def _t(col):
    si = lax.broadcasted_iota(jnp.int32, (_K, _K), 0)
    li = lax.broadcasted_iota(jnp.int32, (_K, _K), 1)
    return jnp.sum(jnp.where(si == li, col, 0.0), axis=0, keepdims=True)


def _kern(cls3_a, flat_a, rois_a, bbox_a, im_a,
          cls3_b, flat_b, rois_b, bbox_b, im_b,
          out_a, out_b, out_idx,
          cand_a, cidx_a, fvec_a, svec_a, g_a, r_a,
          cand_b, cidx_b, fvec_b, svec_b, g_b, r_b):
    wm_a = im_a[0, 1] - 1.0
    hm_a = im_a[0, 0] - 1.0
    ax1, ay1, ax2, ay2, asv = _select_one(
        cls3_a[...], flat_a, rois_a, bbox_a, wm_a, hm_a,
        cand_a, cidx_a, fvec_a, svec_a, g_a, r_a)

    wm_b = im_b[0, 1] - 1.0
    hm_b = im_b[0, 0] - 1.0
    bx1, by1, bx2, by2, bsv = _select_one(
        cls3_b[...], flat_b, rois_b, bbox_b, wm_b, hm_b,
        cand_b, cidx_b, fvec_b, svec_b, g_b, r_b)

    bx1t, by1t, bx2t, by2t = _t(bx1), _t(by1), _t(bx2), _t(by2)
    area_a = (ax2 - ax1 + 1.0) * (ay2 - ay1 + 1.0)
    area_bt = (bx2t - bx1t + 1.0) * (by2t - by1t + 1.0)

    ltx = jnp.maximum(ax1, bx1t)
    lty = jnp.maximum(ay1, by1t)
    rbx = jnp.minimum(ax2, bx2t)
    rby = jnp.minimum(ay2, by2t)
    wi = jnp.maximum(rbx - ltx + 1.0, 0.0)
    hi = jnp.maximum(rby - lty + 1.0, 0.0)
    inter = wi * hi
    iou = inter / (area_a + area_bt - inter)

    m = jnp.max(iou, axis=1, keepdims=True)
    l50 = lax.broadcasted_iota(jnp.int32, (_K, _K), 1)
    idx = jnp.min(jnp.where(iou == m, l50, _BIG), axis=1, keepdims=True)

    box_b = jnp.concatenate([bx1, by1, bx2, by2, bsv], axis=1)
    oh = (l50 == idx).astype(jnp.float32)
    out_a[...] = jnp.concatenate([ax1, ay1, ax2, ay2, asv], axis=1)
    out_b[...] = jnp.dot(oh, box_b, preferred_element_type=jnp.float32)
    out_idx[...] = idx


@functools.partial(jax.jit, static_argnames=())
def _run(rois_a, cls_a, bbox_a, im_a, rois_b, cls_b, bbox_b, im_b):
    sa = cls_a[:, 1:]
    sb = cls_b[:, 1:]
    cls3_a = sa.reshape(_SR, _SC, _SEG)
    flat_a = sa.reshape(_NSEG, _SEG)
    cls3_b = sb.reshape(_SR, _SC, _SEG)
    flat_b = sb.reshape(_NSEG, _SEG)

    vspec = pl.BlockSpec(memory_space=pltpu.VMEM)
    sspec = pl.BlockSpec(memory_space=pltpu.SMEM)
    out = pl.pallas_call(
        _kern,
        out_shape=[
            jax.ShapeDtypeStruct((_K, 5), jnp.float32),
            jax.ShapeDtypeStruct((_K, 5), jnp.float32),
            jax.ShapeDtypeStruct((_K, 1), jnp.int32),
        ],
        in_specs=[vspec, vspec, vspec, vspec, sspec,
                  vspec, vspec, vspec, vspec, sspec],
        out_specs=[vspec, vspec, vspec],
        scratch_shapes=[
            pltpu.VMEM((_K, _SEG), jnp.float32),
            pltpu.VMEM((_K, _SEG), jnp.int32),
            pltpu.VMEM((_K, 1), jnp.int32),
            pltpu.VMEM((_K, 1), jnp.float32),
            pltpu.VMEM((_K, 4 * _C), jnp.float32),
            pltpu.VMEM((_K, 5), jnp.float32),
        ] * 2,
    )(cls3_a, flat_a, rois_a, bbox_a, im_a,
      cls3_b, flat_b, rois_b, bbox_b, im_b)
    box_a, box_b, idx = out
    return box_a, box_b, idx.reshape(_K)


def kernel(rois_A, cls_prob_A, bbox_pred_A, im_info_A,
           rois_B, cls_prob_B, bbox_pred_B, im_info_B):
    return _run(rois_A, cls_prob_A, bbox_pred_A, im_info_A,
                rois_B, cls_prob_B, bbox_pred_B, im_info_B)
